# Initial kernel scaffold; baseline (speedup 1.0000x reference)
import functools

import jax
import jax.numpy as jnp
from jax.experimental import pallas as pl
from jax.experimental.pallas import tpu as pltpu

D_MODEL = 1024
D_FF = 4096
NE = 16
ROUTER_TB = 512
FFN_TB = 256


def _router_body(x_ref, wr_ref, s_ref):
    x = x_ref[...]
    logits = jax.lax.dot_general(x, wr_ref[...], (((1,), (1,)), ((), ())),
                                 preferred_element_type=jnp.float32)
    m = jnp.max(logits, axis=-1, keepdims=True)
    p = jnp.exp(logits - m)
    p = p / jnp.sum(p, axis=-1, keepdims=True)
    ii = jax.lax.broadcasted_iota(jnp.int32, p.shape, 1)
    m1 = jnp.max(p, axis=-1, keepdims=True)
    i1 = jnp.min(jnp.where(p == m1, ii, NE), axis=-1, keepdims=True)
    p2 = jnp.where(ii == i1, -1.0, p)
    m2 = jnp.max(p2, axis=-1, keepdims=True)
    i2 = jnp.min(jnp.where(p2 == m2, ii, NE), axis=-1, keepdims=True)
    denom = m1 + m2
    s = jnp.where(ii == i1, m1 / denom, 0.0) + jnp.where(ii == i2, m2 / denom, 0.0)
    s_ref[...] = s


def _router(xf, Wr, interpret=False):
    n = xf.shape[0]
    ntb = n // ROUTER_TB
    return pl.pallas_call(
        _router_body,
        grid=(ntb,),
        in_specs=[
            pl.BlockSpec((ROUTER_TB, D_MODEL), lambda i: (i, 0)),
            pl.BlockSpec((NE, D_MODEL), lambda i: (0, 0)),
        ],
        out_specs=pl.BlockSpec((ROUTER_TB, NE), lambda i: (i, 0)),
        out_shape=jax.ShapeDtypeStruct((n, NE), jnp.float32),
        interpret=interpret,
    )(xf, Wr)


def _ffn_body(s_ref, x_ref, w1_ref, b1_ref, w2_ref, b2_ref, o_ref, acc_ref):
    e = pl.program_id(0)
    tb = pl.program_id(1)
    x = x_ref[...].astype(jnp.bfloat16)
    h = jax.lax.dot_general(x, w1_ref[0], (((1,), (1,)), ((), ())),
                            preferred_element_type=jnp.float32)
    h = h + b1_ref[0]
    h = jax.nn.gelu(h, approximate=False).astype(jnp.bfloat16)
    y = jax.lax.dot_general(h, w2_ref[0], (((1,), (1,)), ((), ())),
                            preferred_element_type=jnp.float32)
    y = y + b2_ref[0]
    s = s_ref[...]
    ii = jax.lax.broadcasted_iota(jnp.int32, s.shape, 1)
    w_e = jnp.sum(jnp.where(ii == e, s, 0.0), axis=-1, keepdims=True)
    contrib = w_e * y
    sl = pl.ds(tb * FFN_TB, FFN_TB)

    @pl.when(e == 0)
    def _():
        acc_ref[sl, :] = contrib

    @pl.when(e > 0)
    def _():
        acc_ref[sl, :] = acc_ref[sl, :] + contrib

    @pl.when(e == NE - 1)
    def _():
        o_ref[...] = acc_ref[sl, :]


def _ffn(S, xf, W1b, b1, W2b, b2, interpret=False):
    n = xf.shape[0]
    ntb = n // FFN_TB
    return pl.pallas_call(
        _ffn_body,
        grid=(NE, ntb),
        in_specs=[
            pl.BlockSpec((FFN_TB, NE), lambda e, t: (t, 0)),
            pl.BlockSpec((FFN_TB, D_MODEL), lambda e, t: (t, 0)),
            pl.BlockSpec((1, D_FF, D_MODEL), lambda e, t: (e, 0, 0)),
            pl.BlockSpec((1, 1, D_FF), lambda e, t: (e, 0, 0)),
            pl.BlockSpec((1, D_MODEL, D_FF), lambda e, t: (e, 0, 0)),
            pl.BlockSpec((1, 1, D_MODEL), lambda e, t: (e, 0, 0)),
        ],
        out_specs=pl.BlockSpec((FFN_TB, D_MODEL), lambda e, t: (t, 0)),
        out_shape=jax.ShapeDtypeStruct((n, D_MODEL), jnp.float32),
        scratch_shapes=[pltpu.VMEM((n, D_MODEL), jnp.float32)],
        compiler_params=pltpu.CompilerParams(
            dimension_semantics=("arbitrary", "arbitrary")),
        interpret=interpret,
    )(S, xf, W1b, b1, W2b, b2)


def kernel(x, Wr, W1, b1, W2, b2, interpret=False):
    B, T, D = x.shape
    xf = x.reshape(B * T, D)
    S = _router(xf, Wr, interpret=interpret)
    W1b = W1.astype(jnp.bfloat16)
    W2b = W2.astype(jnp.bfloat16)
    b1r = b1.reshape(NE, 1, D_FF)
    b2r = b2.reshape(NE, 1, D_MODEL)
    out = _ffn(S, xf, W1b, b1r, W2b, b2r, interpret=interpret)
    return out.reshape(B, T, D)


# speedup vs baseline: 4.2427x; 4.2427x over previous
import functools

import jax
import jax.numpy as jnp
from jax import lax
from jax.experimental import pallas as pl
from jax.experimental.pallas import tpu as pltpu
from jax.experimental.pallas import tpu_sc as plsc

D_MODEL = 1024
D_FF = 4096
NE = 16
N_TOKENS = 4096
ROUTER_TB = 512
BLK = 256
NB = N_TOKENS * 2 // BLK + NE
NSLOTS = NB * BLK
NW = 32
CHUNK = N_TOKENS // NW


def _router_body(x_ref, wr_ref, s_ref, cnt_ref):
    i = pl.program_id(0)
    x = x_ref[...]
    logits = lax.dot_general(x, wr_ref[...], (((1,), (1,)), ((), ())),
                             preferred_element_type=jnp.float32)
    m = jnp.max(logits, axis=-1, keepdims=True)
    p = jnp.exp(logits - m)
    p = p / jnp.sum(p, axis=-1, keepdims=True)
    ii = lax.broadcasted_iota(jnp.int32, p.shape, 1)
    m1 = jnp.max(p, axis=-1, keepdims=True)
    i1 = jnp.min(jnp.where(p == m1, ii, NE), axis=-1, keepdims=True)
    p2 = jnp.where(ii == i1, -1.0, p)
    m2 = jnp.max(p2, axis=-1, keepdims=True)
    i2 = jnp.min(jnp.where(p2 == m2, ii, NE), axis=-1, keepdims=True)
    denom = m1 + m2
    s = jnp.where(ii == i1, m1 / denom, 0.0) + jnp.where(ii == i2, m2 / denom, 0.0)
    s_ref[...] = s
    active = (s > 0.0).astype(jnp.int32)
    nsub = ROUTER_TB // CHUNK
    parts = [jnp.sum(active[k * CHUNK:(k + 1) * CHUNK], axis=0, keepdims=True)
             for k in range(nsub)]
    cnt_ref[pl.ds(i * nsub, nsub), :] = jnp.concatenate(parts, axis=0)


def _router(xf, Wr):
    n = xf.shape[0]
    ntb = n // ROUTER_TB
    return pl.pallas_call(
        _router_body,
        grid=(ntb,),
        in_specs=[
            pl.BlockSpec((ROUTER_TB, D_MODEL), lambda i: (i, 0)),
            pl.BlockSpec((NE, D_MODEL), lambda i: (0, 0)),
        ],
        out_specs=[
            pl.BlockSpec((ROUTER_TB, NE), lambda i: (i, 0)),
            pl.BlockSpec((NW, NE), lambda i: (0, 0)),
        ],
        out_shape=[
            jax.ShapeDtypeStruct((n, NE), jnp.float32),
            jax.ShapeDtypeStruct((NW, NE), jnp.int32),
        ],
    )(xf, Wr)


def _plan(s_flat, cnt_flat):
    mesh = plsc.VectorSubcoreMesh(core_axis_name="c", subcore_axis_name="s")

    @functools.partial(
        pl.kernel,
        out_type=[
            jax.ShapeDtypeStruct((NSLOTS,), jnp.int32),
            jax.ShapeDtypeStruct((NSLOTS,), jnp.float32),
            jax.ShapeDtypeStruct((2 * NW, 128), jnp.int32),
            jax.ShapeDtypeStruct((NB,), jnp.int32),
        ],
        mesh=mesh,
        compiler_params=pltpu.CompilerParams(needs_layout_passes=False),
        scratch_types=[
            pltpu.VMEM((CHUNK * NE,), jnp.float32),
            pltpu.VMEM((NW * NE,), jnp.int32),
            pltpu.VMEM((2 * CHUNK,), jnp.int32),
            pltpu.VMEM((2 * CHUNK,), jnp.float32),
            pltpu.VMEM((2, 128), jnp.int32),
            pltpu.VMEM((NE,), jnp.int32),
            pltpu.VMEM((NB,), jnp.int32),
            pltpu.SemaphoreType.DMA,
        ],
    )
    def k(s_hbm, cnt_hbm, tok_hbm, w_hbm, so_hbm, be_hbm,
          s_v, cnt_v, tbuf, wbuf, didx2, starts_v, bexp_v, sem):
        wid = lax.axis_index("s") * 2 + lax.axis_index("c")
        pltpu.sync_copy(cnt_hbm, cnt_v)
        pltpu.sync_copy(s_hbm.at[pl.ds(wid * CHUNK * NE, CHUNK * NE)], s_v)

        zero = jnp.zeros((NE,), jnp.int32)
        total = zero
        prefix = zero
        for c in range(NW):
            row = cnt_v[pl.ds(c * NE, NE)]
            total = total + row
            sel = jnp.full((NE,), c, jnp.int32) < wid
            prefix = prefix + jnp.where(sel, row, 0)
        padded = ((total + (BLK - 1)) >> 8) << 8
        lane16 = lax.iota(jnp.int32, NE)
        xsc = padded
        for sh in (1, 2, 4, 8):
            starts_v[...] = xsc
            g = plsc.load_gather(starts_v, [jnp.maximum(lane16 - sh, 0)])
            xsc = xsc + jnp.where(lane16 >= sh, g, 0)
        starts = xsc - padded
        starts_v[...] = starts
        mybase = starts + prefix

        lane = lax.iota(jnp.int32, NE)

        def body(i, mb):
            row = s_v[pl.ds(i * NE, NE)]
            m = row > 0.0
            mi = jnp.where(m, 1, 0)
            ffs = plsc.all_reduce_ffs(m)
            kk = jnp.where(lane == ffs, 0, 1)
            idx2 = kk + 2 * i
            plsc.store_scatter(didx2, [idx2 >> 7, idx2 & 127], mb, mask=m)
            plsc.store_scatter(wbuf, [idx2], row, mask=m)
            return mb + mi

        lax.fori_loop(0, CHUNK, body, mybase)

        iota = lax.iota(jnp.int32, NE)
        for v in range(2 * CHUNK // NE):
            tbuf[pl.ds(v * NE, NE)] = wid * CHUNK + ((iota + v * NE) >> 1)

        for h in range(2):
            pltpu.async_copy(tbuf.at[pl.ds(h * 128, 128)],
                             tok_hbm.at[didx2.at[h]], sem).wait()
            pltpu.async_copy(wbuf.at[pl.ds(h * 128, 128)],
                             w_hbm.at[didx2.at[h]], sem).wait()
        pltpu.sync_copy(didx2, so_hbm.at[pl.ds(2 * wid, 2)])

        @pl.when(wid == 0)
        def _():
            for j in range(NB // NE):
                bv = (lax.iota(jnp.int32, NE) + NE * j) * BLK
                acc = jnp.zeros((NE,), jnp.int32)
                for e in range(NE):
                    se = plsc.load_gather(starts_v,
                                          [jnp.full((NE,), e, jnp.int32)])
                    acc = acc + jnp.where(bv >= se, 1, 0)
                bexp_v[pl.ds(NE * j, NE)] = acc - 1
            pltpu.sync_copy(bexp_v, be_hbm)

    return k(s_flat, cnt_flat)


def _gather(tok, xf):
    mesh = plsc.VectorSubcoreMesh(core_axis_name="c", subcore_axis_name="s")
    rows_per_w = NSLOTS // NW
    g_rows = 96

    @functools.partial(
        pl.kernel,
        out_type=jax.ShapeDtypeStruct((NSLOTS, D_MODEL), jnp.float32),
        mesh=mesh,
        compiler_params=pltpu.CompilerParams(needs_layout_passes=False),
        scratch_types=[
            pltpu.VMEM((rows_per_w,), jnp.int32),
            pltpu.VMEM((g_rows, D_MODEL), jnp.float32),
            pltpu.SemaphoreType.DMA,
        ],
    )
    def k(tok_hbm, x_hbm, xd_hbm, idx_v, rows_v, sem):
        wid = lax.axis_index("s") * 2 + lax.axis_index("c")
        base = wid * rows_per_w
        pltpu.sync_copy(tok_hbm.at[pl.ds(base, rows_per_w)], idx_v)
        for v in range(rows_per_w // NE):
            sl = pl.ds(v * NE, NE)
            idx_v[sl] = jnp.clip(idx_v[sl], 0, N_TOKENS - 1)
        for g in range(rows_per_w // g_rows):
            pltpu.async_copy(x_hbm.at[idx_v.at[pl.ds(g * g_rows, g_rows)]],
                             rows_v, sem).wait()
            pltpu.sync_copy(rows_v, xd_hbm.at[pl.ds(base + g * g_rows, g_rows)])

    return k(tok, xf)


def _gelu(h):
    return 0.5 * h * (1.0 + lax.erf(h * 0.7071067811865476))


def _ffn_body(be_ref, w_ref, x_ref, w1_ref, b1_ref, w2_ref, b2_ref, o_ref):
    x = x_ref[...].astype(jnp.bfloat16)
    h = lax.dot_general(x, w1_ref[0], (((1,), (1,)), ((), ())),
                        preferred_element_type=jnp.float32)
    h = _gelu(h + b1_ref[0]).astype(jnp.bfloat16)
    y = lax.dot_general(h, w2_ref[0], (((1,), (1,)), ((), ())),
                        preferred_element_type=jnp.float32)
    o_ref[...] = (y + b2_ref[0]) * w_ref[...]


def _ffn(be, wsl, xd, W1b, b1r, W2b, b2r):
    grid_spec = pltpu.PrefetchScalarGridSpec(
        num_scalar_prefetch=1,
        grid=(NB,),
        in_specs=[
            pl.BlockSpec((BLK, 1), lambda i, be: (i, 0)),
            pl.BlockSpec((BLK, D_MODEL), lambda i, be: (i, 0)),
            pl.BlockSpec((1, D_FF, D_MODEL), lambda i, be: (be[i], 0, 0)),
            pl.BlockSpec((1, 1, D_FF), lambda i, be: (be[i], 0, 0)),
            pl.BlockSpec((1, D_MODEL, D_FF), lambda i, be: (be[i], 0, 0)),
            pl.BlockSpec((1, 1, D_MODEL), lambda i, be: (be[i], 0, 0)),
        ],
        out_specs=pl.BlockSpec((BLK, D_MODEL), lambda i, be: (i, 0)),
    )
    return pl.pallas_call(
        _ffn_body,
        grid_spec=grid_spec,
        out_shape=jax.ShapeDtypeStruct((NSLOTS, D_MODEL), jnp.float32),
        compiler_params=pltpu.CompilerParams(
            dimension_semantics=("arbitrary",)),
    )(be, wsl, xd, W1b, b1r, W2b, b2r)


def _combine(so, yb):
    mesh = plsc.VectorSubcoreMesh(core_axis_name="c", subcore_axis_name="s")
    ct = 32

    @functools.partial(
        pl.kernel,
        out_type=jax.ShapeDtypeStruct((N_TOKENS * D_MODEL,), jnp.float32),
        mesh=mesh,
        compiler_params=pltpu.CompilerParams(needs_layout_passes=False),
        scratch_types=[
            pltpu.VMEM((2, 128), jnp.int32),
            pltpu.VMEM((2 * ct, D_MODEL), jnp.float32),
            pltpu.VMEM((ct * D_MODEL,), jnp.float32),
            pltpu.SemaphoreType.DMA,
        ],
    )
    def k(so_hbm, y_hbm, out_hbm, so_v, rows_v, out_v, sem):
        wid = lax.axis_index("s") * 2 + lax.axis_index("c")
        pltpu.sync_copy(so_hbm.at[pl.ds(2 * wid, 2)], so_v)
        for c in range(CHUNK // ct):
            r, off = divmod(2 * ct * c, 128)
            pltpu.async_copy(y_hbm.at[so_v.at[r, pl.ds(off, 2 * ct)]],
                             rows_v, sem).wait()

            def tok_body(j, _):
                for v in range(D_MODEL // NE):
                    sl = pl.ds(v * NE, NE)
                    a = rows_v[2 * j, sl]
                    b = rows_v[2 * j + 1, sl]
                    out_v[pl.ds(j * D_MODEL + v * NE, NE)] = a + b
                return 0

            lax.fori_loop(0, ct, tok_body, 0)
            pltpu.sync_copy(
                out_v,
                out_hbm.at[pl.ds((wid * CHUNK + c * ct) * D_MODEL,
                                 ct * D_MODEL)])

    return k(so, yb)


def kernel(x, Wr, W1, b1, W2, b2):
    B, T, D = x.shape
    xf = x.reshape(B * T, D)
    S, counts = _router(xf, Wr)
    tok, wsl, so, be = _plan(S.reshape(-1), counts.reshape(-1))
    xd = _gather(tok, xf)
    W1b = W1.astype(jnp.bfloat16)
    W2b = W2.astype(jnp.bfloat16)
    b1r = b1.reshape(NE, 1, D_FF)
    b2r = b2.reshape(NE, 1, D_MODEL)
    yb = _ffn(be, wsl.reshape(NSLOTS, 1), xd, W1b, b1r, W2b, b2r)
    out = _combine(so, yb)
    return out.reshape(B, T, D)
